# baseline (speedup 1.0000x reference)
import functools

import jax
import jax.numpy as jnp
from jax import lax
from jax.experimental import pallas as pl
from jax.experimental.pallas import tpu as pltpu

_ROT1 = (13, 15, 26, 6)
_ROT2 = (17, 29, 16, 24)
_KS1 = 42
_KS2 = 0x1BD11BDA ^ _KS1


def _rotl(x, r):
    return lax.shift_left(x, jnp.uint32(r)) | lax.shift_right_logical(
        x, jnp.uint32(32 - r))


def _round(x0, x1, r):
    x0 = x0 + x1
    x1 = _rotl(x1, r) ^ x0
    return x0, x1


def _threefry_bits(x1):
    x0 = x1
    x1 = _rotl(x1, _ROT1[0]) ^ x0
    for r in _ROT1[1:]:
        x0, x1 = _round(x0, x1, r)
    x0 = x0 + jnp.uint32(_KS1)
    x1 = x1 + jnp.uint32(_KS2 + 1)
    for r in _ROT2:
        x0, x1 = _round(x0, x1, r)
    x0 = x0 + jnp.uint32(_KS2)
    x1 = x1 + jnp.uint32(2)
    for r in _ROT1:
        x0, x1 = _round(x0, x1, r)
    x1 = x1 + jnp.uint32(_KS1 + 3)
    for r in _ROT2:
        x0, x1 = _round(x0, x1, r)
    x0 = x0 + jnp.uint32(_KS1)
    x1 = x1 + jnp.uint32(_KS2 + 4)
    for r in _ROT1:
        x0, x1 = _round(x0, x1, r)
    x0 = x0 + jnp.uint32(_KS2)
    x1 = x1 + jnp.uint32(5)
    return x0 ^ x1


def _gumbel_argmax_kernel(logits_ref, out_ref, best_val, best_idx,
                          *, ncol, block_cols, slice_cols, nblocks):
    j = pl.program_id(0)
    nrow = logits_ref.shape[0]
    nslices = block_cols // slice_cols
    col0 = j * block_cols

    col_iota = lax.broadcasted_iota(jnp.int32, (nrow, slice_cols), 1)
    base = lax.broadcasted_iota(jnp.uint32, (nrow, slice_cols), 0) * \
        jnp.uint32(ncol) + col_iota.astype(jnp.uint32)

    def block_reduce(masked):
        bv = None
        bi = None
        for s in range(nslices):
            off = col0 + s * slice_cols
            bits = _threefry_bits(base + (off + _KS1).astype(jnp.uint32))
            fbits = lax.shift_right_logical(bits, jnp.uint32(9)) | \
                jnp.uint32(0x3F800000)
            u = lax.bitcast_convert_type(fbits, jnp.float32) - jnp.float32(1.0)
            u = u + jnp.float32(1e-20)
            nlu = -jnp.log(u)
            v = logits_ref[:, s * slice_cols:(s + 1) * slice_cols] - \
                jnp.log(nlu)
            if masked:
                v = jnp.where(col_iota < ncol - off, v, -jnp.inf)
            m = jnp.max(v, axis=1, keepdims=True)
            i = jnp.min(jnp.where(v == m, col_iota, jnp.int32(0x7FFFFFFF)),
                        axis=1, keepdims=True) + off
            if s == 0:
                bv, bi = m, i
            else:
                upd = m > bv
                bv = jnp.where(upd, m, bv)
                bi = jnp.where(upd, i, bi)
        return bv, bi

    def accumulate(bv, bi, write_out):
        @pl.when(j == 0)
        def _():
            best_val[...] = bv
            best_idx[...] = bi

        @pl.when(j > 0)
        def _():
            prev_v = best_val[...]
            prev_i = best_idx[...]
            upd = bv > prev_v
            best_val[...] = jnp.where(upd, bv, prev_v)
            best_idx[...] = jnp.where(upd, bi, prev_i)

        if write_out:
            out_ref[...] = best_idx[...]

    @pl.when(j < nblocks - 1)
    def _():
        bv, bi = block_reduce(masked=False)
        accumulate(bv, bi, write_out=False)

    @pl.when(j == nblocks - 1)
    def _():
        bv, bi = block_reduce(masked=True)
        accumulate(bv, bi, write_out=True)


def kernel(logits):
    nrow, ncol = logits.shape
    block_cols = 16384
    slice_cols = 1024
    nblocks = pl.cdiv(ncol, block_cols)

    body = functools.partial(_gumbel_argmax_kernel, ncol=ncol,
                             block_cols=block_cols, slice_cols=slice_cols,
                             nblocks=nblocks)
    out = pl.pallas_call(
        body,
        grid=(nblocks,),
        in_specs=[pl.BlockSpec((nrow, block_cols), lambda j: (0, j))],
        out_specs=pl.BlockSpec((nrow, 1), lambda j: (0, 0)),
        out_shape=jax.ShapeDtypeStruct((nrow, 1), jnp.int32),
        scratch_shapes=[
            pltpu.VMEM((nrow, 1), jnp.float32),
            pltpu.VMEM((nrow, 1), jnp.int32),
        ],
    )(logits)
    return out.reshape(nrow).astype(jnp.int64)


# speedup vs baseline: 2.0727x; 2.0727x over previous
import functools

import jax
import jax.numpy as jnp
from jax import lax
from jax.experimental import pallas as pl
from jax.experimental.pallas import tpu as pltpu

_ROT1 = (13, 15, 26, 6)
_ROT2 = (17, 29, 16, 24)
_KS1 = 42
_KS2 = 0x1BD11BDA ^ _KS1


def _rotl(x, r):
    return lax.shift_left(x, jnp.uint32(r)) | lax.shift_right_logical(
        x, jnp.uint32(32 - r))


def _round(x0, x1, r):
    x0 = x0 + x1
    x1 = _rotl(x1, r) ^ x0
    return x0, x1


def _threefry_bits(x1):
    x0 = x1
    x1 = _rotl(x1, _ROT1[0]) ^ x0
    for r in _ROT1[1:]:
        x0, x1 = _round(x0, x1, r)
    x0 = x0 + jnp.uint32(_KS1)
    x1 = x1 + jnp.uint32(_KS2 + 1)
    for r in _ROT2:
        x0, x1 = _round(x0, x1, r)
    x0 = x0 + jnp.uint32(_KS2)
    x1 = x1 + jnp.uint32(2)
    for r in _ROT1:
        x0, x1 = _round(x0, x1, r)
    x1 = x1 + jnp.uint32(_KS1 + 3)
    for r in _ROT2:
        x0, x1 = _round(x0, x1, r)
    x0 = x0 + jnp.uint32(_KS1)
    x1 = x1 + jnp.uint32(_KS2 + 4)
    for r in _ROT1:
        x0, x1 = _round(x0, x1, r)
    x0 = x0 + jnp.uint32(_KS2)
    x1 = x1 + jnp.uint32(5)
    return x0 ^ x1


def _gumbel_argmax_kernel(logits_ref, out_ref, best_val, best_idx,
                          *, ncol, block_cols, slice_cols, nblocks):
    j = pl.program_id(0)
    nrow = logits_ref.shape[0]
    nslices = block_cols // slice_cols
    col0 = j * block_cols

    col_iota = lax.broadcasted_iota(jnp.int32, (nrow, slice_cols), 1)
    base = lax.broadcasted_iota(jnp.uint32, (nrow, slice_cols), 0) * \
        jnp.uint32(ncol) + col_iota.astype(jnp.uint32)

    def block_reduce(masked):
        bv = None
        bi = None
        for s in range(nslices):
            off = col0 + s * slice_cols
            bits = _threefry_bits(base + (off + _KS1).astype(jnp.uint32))
            fbits = lax.shift_right_logical(bits, jnp.uint32(9)) | \
                jnp.uint32(0x3F800000)
            u = lax.bitcast_convert_type(fbits, jnp.float32) - jnp.float32(1.0)
            u = u + jnp.float32(1e-20)
            nlu = -jnp.log(u)
            v = logits_ref[:, s * slice_cols:(s + 1) * slice_cols] - \
                jnp.log(nlu)
            if masked:
                v = jnp.where(col_iota < ncol - off, v, -jnp.inf)
            m = jnp.max(v, axis=1, keepdims=True)
            i = jnp.min(jnp.where(v == m, col_iota, jnp.int32(0x7FFFFFFF)),
                        axis=1, keepdims=True) + off
            if s == 0:
                bv, bi = m, i
            else:
                upd = m > bv
                bv = jnp.where(upd, m, bv)
                bi = jnp.where(upd, i, bi)
        return bv, bi

    bv, bi = block_reduce(masked=True)

    @pl.when(j == 0)
    def _():
        best_val[...] = bv
        best_idx[...] = bi

    @pl.when(j > 0)
    def _():
        prev_v = best_val[...]
        prev_i = best_idx[...]
        upd = bv > prev_v
        best_val[...] = jnp.where(upd, bv, prev_v)
        best_idx[...] = jnp.where(upd, bi, prev_i)

    @pl.when(j == nblocks - 1)
    def _():
        out_ref[...] = best_idx[...]


def kernel(logits):
    nrow, ncol = logits.shape
    block_cols = 16384
    slice_cols = 1024
    nblocks = pl.cdiv(ncol, block_cols)

    body = functools.partial(_gumbel_argmax_kernel, ncol=ncol,
                             block_cols=block_cols, slice_cols=slice_cols,
                             nblocks=nblocks)
    out = pl.pallas_call(
        body,
        grid=(nblocks,),
        in_specs=[pl.BlockSpec((nrow, block_cols), lambda j: (0, j))],
        out_specs=pl.BlockSpec((nrow, 1), lambda j: (0, 0)),
        out_shape=jax.ShapeDtypeStruct((nrow, 1), jnp.int32),
        scratch_shapes=[
            pltpu.VMEM((nrow, 1), jnp.float32),
            pltpu.VMEM((nrow, 1), jnp.int32),
        ],
    )(logits)
    return out.reshape(nrow).astype(jnp.int64)


# speedup vs baseline: 2.1474x; 1.0360x over previous
import functools

import jax
import jax.numpy as jnp
from jax import lax
from jax.experimental import pallas as pl
from jax.experimental.pallas import tpu as pltpu

_ROT1 = (13, 15, 26, 6)
_ROT2 = (17, 29, 16, 24)
_KS1 = 42
_KS2 = 0x1BD11BDA ^ _KS1
_BIG = 0x7FFFFFFF


def _rotl(x, r):
    return lax.shift_left(x, jnp.uint32(r)) | lax.shift_right_logical(
        x, jnp.uint32(32 - r))


def _round(x0, x1, r):
    x0 = x0 + x1
    x1 = _rotl(x1, r) ^ x0
    return x0, x1


def _threefry_bits(x1):
    x0 = x1
    x1 = _rotl(x1, _ROT1[0]) ^ x0
    for r in _ROT1[1:]:
        x0, x1 = _round(x0, x1, r)
    x0 = x0 + jnp.uint32(_KS1)
    x1 = x1 + jnp.uint32(_KS2 + 1)
    for r in _ROT2:
        x0, x1 = _round(x0, x1, r)
    x0 = x0 + jnp.uint32(_KS2)
    x1 = x1 + jnp.uint32(2)
    for r in _ROT1:
        x0, x1 = _round(x0, x1, r)
    x1 = x1 + jnp.uint32(_KS1 + 3)
    for r in _ROT2:
        x0, x1 = _round(x0, x1, r)
    x0 = x0 + jnp.uint32(_KS1)
    x1 = x1 + jnp.uint32(_KS2 + 4)
    for r in _ROT1:
        x0, x1 = _round(x0, x1, r)
    x0 = x0 + jnp.uint32(_KS2)
    x1 = x1 + jnp.uint32(5)
    return x0 ^ x1


def _gumbel_slice(logits_slice, base, off):
    bits = _threefry_bits(base + jnp.uint32(off + _KS1))
    fbits = lax.shift_right_logical(bits, jnp.uint32(9)) | \
        jnp.uint32(0x3F800000)
    u = lax.bitcast_convert_type(fbits, jnp.float32) - jnp.float32(1.0)
    u = u + jnp.float32(1e-20)
    nlu = -jnp.log(u)
    return logits_slice - jnp.log(nlu)


def _main_kernel(logits_ref, val_ref, idx_ref, best_val, best_idx,
                 *, ncol, block_cols, slice_cols, nblocks):
    j = pl.program_id(0)
    nrow = logits_ref.shape[0]
    nslices = block_cols // slice_cols
    col0 = j * block_cols

    col_iota = lax.broadcasted_iota(jnp.int32, (nrow, slice_cols), 1)
    base = lax.broadcasted_iota(jnp.uint32, (nrow, slice_cols), 0) * \
        jnp.uint32(ncol) + col_iota.astype(jnp.uint32)

    acc = None
    idxacc = None
    for s in range(nslices):
        v = _gumbel_slice(
            logits_ref[:, s * slice_cols:(s + 1) * slice_cols],
            base, col0 + s * slice_cols)
        if s == 0:
            acc = v
            idxacc = jnp.zeros((nrow, slice_cols), jnp.int32)
        else:
            upd = v > acc
            acc = jnp.where(upd, v, acc)
            idxacc = jnp.where(upd, jnp.int32(s * slice_cols), idxacc)

    m = jnp.max(acc, axis=1, keepdims=True)
    enc = idxacc + col_iota
    bi = jnp.min(jnp.where(acc == m, enc, jnp.int32(_BIG)),
                 axis=1, keepdims=True) + col0
    bv = m

    @pl.when(j == 0)
    def _():
        best_val[...] = bv
        best_idx[...] = bi

    @pl.when(j > 0)
    def _():
        prev_v = best_val[...]
        prev_i = best_idx[...]
        upd = bv > prev_v
        best_val[...] = jnp.where(upd, bv, prev_v)
        best_idx[...] = jnp.where(upd, bi, prev_i)

    @pl.when(j == nblocks - 1)
    def _():
        val_ref[...] = best_val[...]
        idx_ref[...] = best_idx[...]


def _tail_kernel(tail_ref, mval_ref, midx_ref, out_ref, *, ncol, tail_col0):
    nrow, w = tail_ref.shape
    col_iota = lax.broadcasted_iota(jnp.int32, (nrow, w), 1)
    base = lax.broadcasted_iota(jnp.uint32, (nrow, w), 0) * \
        jnp.uint32(ncol) + col_iota.astype(jnp.uint32) + jnp.uint32(tail_col0)
    v = _gumbel_slice(tail_ref[...], base, 0)
    m = jnp.max(v, axis=1, keepdims=True)
    i = jnp.min(jnp.where(v == m, col_iota, jnp.int32(_BIG)),
                axis=1, keepdims=True) + tail_col0
    mv = mval_ref[...]
    upd = m > mv
    out_ref[...] = jnp.where(upd, i, midx_ref[...])


def kernel(logits):
    nrow, ncol = logits.shape
    block_cols = 16384
    slice_cols = 1024
    nblocks = ncol // block_cols
    main_cols = nblocks * block_cols
    tail_len = ncol - main_cols

    body = functools.partial(_main_kernel, ncol=ncol, block_cols=block_cols,
                             slice_cols=slice_cols, nblocks=nblocks)
    mval, midx = pl.pallas_call(
        body,
        grid=(nblocks,),
        in_specs=[pl.BlockSpec((nrow, block_cols), lambda j: (0, j))],
        out_specs=[
            pl.BlockSpec((nrow, 1), lambda j: (0, 0)),
            pl.BlockSpec((nrow, 1), lambda j: (0, 0)),
        ],
        out_shape=[
            jax.ShapeDtypeStruct((nrow, 1), jnp.float32),
            jax.ShapeDtypeStruct((nrow, 1), jnp.int32),
        ],
        scratch_shapes=[
            pltpu.VMEM((nrow, 1), jnp.float32),
            pltpu.VMEM((nrow, 1), jnp.int32),
        ],
    )(logits)

    if tail_len == 0:
        return midx.reshape(nrow).astype(jnp.int64)

    pad = (-tail_len) % 128
    tail = logits[:, main_cols:]
    if pad:
        tail = jnp.pad(tail, ((0, 0), (0, pad)),
                       constant_values=-jnp.inf)
    tail_body = functools.partial(_tail_kernel, ncol=ncol,
                                  tail_col0=main_cols)
    out = pl.pallas_call(
        tail_body,
        out_shape=jax.ShapeDtypeStruct((nrow, 1), jnp.int32),
    )(tail, mval, midx)
    return out.reshape(nrow).astype(jnp.int64)
